# baseline (speedup 1.0000x reference)
import math

import numpy as np
import jax
import jax.numpy as jnp
from jax.experimental import pallas as pl
from jax.experimental.pallas import tpu as pltpu

M = 65536
C = 1024
BM = 512
ALPHA = 0.7
BETA = 0.3
EPS = 1e-06
K = 13108


def _xla_rowsum(a):
    t = a[:, 0:128]
    for c in range(1, 8):
        t = t + a[:, 128 * c:128 * (c + 1)]
    u = t[:, 0:8]
    for k in range(1, 16):
        u = u + t[:, 8 * k:8 * (k + 1)]
    u = u[:, :4] + u[:, 4:]
    u = u[:, :2] + u[:, 2:]
    return u[:, 0] + u[:, 1]


def _entropy_kernel(x_ref, out_ref):
    x = x_ref[...]
    x = jnp.nan_to_num(x, nan=0.0, posinf=0.0, neginf=0.0)
    m = jnp.max(x, axis=1, keepdims=True)
    e = jnp.exp(x - m)
    z = _xla_rowsum(e)
    p = e / z[:, None]
    lp = jnp.log(p + EPS)
    ent = -_xla_rowsum(p * lp)
    out_ref[0, 0, :] = ent


ENT_SCALE = float(np.float32(np.float32(ALPHA) / np.float32(math.log(C) + EPS)))


def _combine_kernel(ent_ref, geo_ref, out_ref):
    g = geo_ref[...]
    g = jnp.nan_to_num(g, nan=0.0, posinf=0.0, neginf=0.0)
    gmin = jnp.min(g)
    gmax = jnp.max(g)
    gn = (g - gmin) / (gmax - gmin + EPS)
    out_ref[...] = ent_ref[...] * ENT_SCALE + BETA * gn


def _scores2d(coarse_logits, handcrafted_features):
    nb = M // BM
    ent = pl.pallas_call(
        _entropy_kernel,
        grid=(nb,),
        in_specs=[pl.BlockSpec((BM, C), lambda i: (i, 0))],
        out_specs=pl.BlockSpec((1, 1, BM), lambda i: (i, 0, 0)),
        out_shape=jax.ShapeDtypeStruct((nb, 1, BM), jnp.float32),
    )(coarse_logits)
    ent2d = ent.reshape(M // 128, 128)
    geo2d = handcrafted_features[:, 2].reshape(M // 128, 128)
    total = pl.pallas_call(
        _combine_kernel,
        in_specs=[
            pl.BlockSpec((M // 128, 128), lambda: (0, 0)),
            pl.BlockSpec((M // 128, 128), lambda: (0, 0)),
        ],
        out_specs=pl.BlockSpec((M // 128, 128), lambda: (0, 0)),
        out_shape=jax.ShapeDtypeStruct((M // 128, 128), jnp.float32),
    )(ent2d, geo2d)
    return total


SR = 512
SC_ = 128
KROWS = 104


def _sort_kernel(sc_ref, idx_out_ref, key_scr, idx_scr):
    i = pl.program_id(0)
    r = jax.lax.broadcasted_iota(jnp.int32, (SR, SC_), 0)
    c = jax.lax.broadcasted_iota(jnp.int32, (SR, SC_), 1)

    @pl.when(i == 0)
    def _init():
        key_scr[...] = sc_ref[...]
        idx_scr[...] = r * SC_ + c

    k = jnp.int32(2) << i
    kr = k >> 7
    asc = ((c & k) == 0) & ((r & kr) == 0)

    for t in range(15, -1, -1):
        j = 1 << t

        @pl.when(2 * j <= k)
        def _stage(j=j):
            mk = key_scr[...]
            mi = idx_scr[...]
            if j >= SC_:
                d = j // SC_
                pk = jnp.where((r & d) != 0,
                               pltpu.roll(mk, d, axis=0),
                               pltpu.roll(mk, SR - d, axis=0))
                pi = jnp.where((r & d) != 0,
                               pltpu.roll(mi, d, axis=0),
                               pltpu.roll(mi, SR - d, axis=0))
                mybit = (r & d) != 0
            else:
                pk = jnp.where((c & j) != 0,
                               pltpu.roll(mk, j, axis=1),
                               pltpu.roll(mk, SC_ - j, axis=1))
                pi = jnp.where((c & j) != 0,
                               pltpu.roll(mi, j, axis=1),
                               pltpu.roll(mi, SC_ - j, axis=1))
                mybit = (c & j) != 0
            less_mp = (mk > pk) | ((mk == pk) & (mi < pi))
            takes_min = asc != mybit
            sel_mine = takes_min == less_mp
            key_scr[...] = jnp.where(sel_mine, mk, pk)
            idx_scr[...] = jnp.where(sel_mine, mi, pi)

    @pl.when(i == 15)
    def _emit():
        idx_out_ref[...] = idx_scr[0:KROWS, :]


def _topk_sort(scores2d):
    idx = pl.pallas_call(
        _sort_kernel,
        grid=(16,),
        in_specs=[pl.BlockSpec((SR, SC_), lambda i: (0, 0))],
        out_specs=pl.BlockSpec((KROWS, SC_), lambda i: (0, 0)),
        out_shape=jax.ShapeDtypeStruct((KROWS, SC_), jnp.int32),
        scratch_shapes=[
            pltpu.VMEM((SR, SC_), jnp.float32),
            pltpu.VMEM((SR, SC_), jnp.int32),
        ],
    )(scores2d)
    return idx.reshape(KROWS * SC_)[:K]


def kernel(coarse_logits, handcrafted_features):
    total2d = _scores2d(coarse_logits, handcrafted_features)
    hard_sp_indices = _topk_sort(total2d)
    return (hard_sp_indices, total2d.reshape(M))


# speedup vs baseline: 2.2106x; 2.2106x over previous
import math

import numpy as np
import jax
import jax.numpy as jnp
from jax.experimental import pallas as pl
from jax.experimental.pallas import tpu as pltpu

M = 65536
C = 1024
BM = 512
ALPHA = 0.7
BETA = 0.3
EPS = 1e-06
K = 13108


def _xla_rowsum_lanes(a):
    nr = a.shape[0]
    t = a[:, 0:128]
    for c in range(1, 8):
        t = t + a[:, 128 * c:128 * (c + 1)]
    zs = []
    for g in range(nr // 128):
        tt = t[128 * g:128 * (g + 1), :].T
        u = tt[0:8, :]
        for k in range(1, 16):
            u = u + tt[8 * k:8 * (k + 1), :]
        u = u[0:4, :] + u[4:8, :]
        u = u[0:2, :] + u[2:4, :]
        zs.append(u[0:1, :] + u[1:2, :])
    return jnp.concatenate(zs, axis=0)


def _rows_to_col(zl, nr):
    cols = [zl[g:g + 1, :].reshape(128, 1) for g in range(nr // 128)]
    return jnp.concatenate(cols, axis=0)


def _entropy_kernel(x_ref, out_ref):
    x = x_ref[...]
    x = jnp.nan_to_num(x, nan=0.0, posinf=0.0, neginf=0.0)
    m = jnp.max(x, axis=1, keepdims=True)
    e = jnp.exp(x - m)
    z = _rows_to_col(_xla_rowsum_lanes(e), BM)
    p = e / z
    lp = jnp.log(p + EPS)
    ent = -_xla_rowsum_lanes(p * lp)
    out_ref[0, 0, :] = ent.reshape(BM)


ENT_SCALE = float(np.float32(np.float32(ALPHA) / np.float32(math.log(C) + EPS)))


def _combine_kernel(ent_ref, geo_ref, out_ref):
    g = geo_ref[...]
    g = jnp.nan_to_num(g, nan=0.0, posinf=0.0, neginf=0.0)
    gmin = jnp.min(g)
    gmax = jnp.max(g)
    gn = (g - gmin) / (gmax - gmin + EPS)
    out_ref[...] = ent_ref[...] * ENT_SCALE + BETA * gn


def _scores2d(coarse_logits, handcrafted_features):
    nb = M // BM
    ent = pl.pallas_call(
        _entropy_kernel,
        grid=(nb,),
        in_specs=[pl.BlockSpec((BM, C), lambda i: (i, 0))],
        out_specs=pl.BlockSpec((1, 1, BM), lambda i: (i, 0, 0)),
        out_shape=jax.ShapeDtypeStruct((nb, 1, BM), jnp.float32),
    )(coarse_logits)
    ent2d = ent.reshape(M // 128, 128)
    geo2d = handcrafted_features[:, 2].reshape(M // 128, 128)
    total = pl.pallas_call(
        _combine_kernel,
        in_specs=[
            pl.BlockSpec((M // 128, 128), lambda: (0, 0)),
            pl.BlockSpec((M // 128, 128), lambda: (0, 0)),
        ],
        out_specs=pl.BlockSpec((M // 128, 128), lambda: (0, 0)),
        out_shape=jax.ShapeDtypeStruct((M // 128, 128), jnp.float32),
    )(ent2d, geo2d)
    return total


SR = 512
SC_ = 128
KROWS = 104


def _sort_kernel(sc_ref, idx_out_ref, key_scr, idx_scr):
    i = pl.program_id(0)
    r = jax.lax.broadcasted_iota(jnp.int32, (SR, SC_), 0)
    c = jax.lax.broadcasted_iota(jnp.int32, (SR, SC_), 1)

    @pl.when(i == 0)
    def _init():
        key_scr[...] = sc_ref[...]
        idx_scr[...] = r * SC_ + c

    k = jnp.int32(2) << i
    kr = k >> 7
    asc = ((c & k) == 0) & ((r & kr) == 0)

    for t in range(15, -1, -1):
        j = 1 << t

        @pl.when(2 * j <= k)
        def _stage(j=j):
            mk = key_scr[...]
            mi = idx_scr[...]
            if j >= SC_:
                d = j // SC_
                pk = jnp.where((r & d) != 0,
                               pltpu.roll(mk, d, axis=0),
                               pltpu.roll(mk, SR - d, axis=0))
                pi = jnp.where((r & d) != 0,
                               pltpu.roll(mi, d, axis=0),
                               pltpu.roll(mi, SR - d, axis=0))
                mybit = (r & d) != 0
            else:
                pk = jnp.where((c & j) != 0,
                               pltpu.roll(mk, j, axis=1),
                               pltpu.roll(mk, SC_ - j, axis=1))
                pi = jnp.where((c & j) != 0,
                               pltpu.roll(mi, j, axis=1),
                               pltpu.roll(mi, SC_ - j, axis=1))
                mybit = (c & j) != 0
            less_mp = (mk > pk) | ((mk == pk) & (mi < pi))
            takes_min = asc != mybit
            sel_mine = takes_min == less_mp
            key_scr[...] = jnp.where(sel_mine, mk, pk)
            idx_scr[...] = jnp.where(sel_mine, mi, pi)

    @pl.when(i == 15)
    def _emit():
        idx_out_ref[...] = idx_scr[0:KROWS, :]


def _topk_sort(scores2d):
    idx = pl.pallas_call(
        _sort_kernel,
        grid=(16,),
        in_specs=[pl.BlockSpec((SR, SC_), lambda i: (0, 0))],
        out_specs=pl.BlockSpec((KROWS, SC_), lambda i: (0, 0)),
        out_shape=jax.ShapeDtypeStruct((KROWS, SC_), jnp.int32),
        scratch_shapes=[
            pltpu.VMEM((SR, SC_), jnp.float32),
            pltpu.VMEM((SR, SC_), jnp.int32),
        ],
    )(scores2d)
    return idx.reshape(KROWS * SC_)[:K]


def kernel(coarse_logits, handcrafted_features):
    total2d = _scores2d(coarse_logits, handcrafted_features)
    hard_sp_indices = _topk_sort(total2d)
    return (hard_sp_indices, total2d.reshape(M))


# speedup vs baseline: 2.2460x; 1.0160x over previous
import math

import numpy as np
import jax
import jax.numpy as jnp
from jax.experimental import pallas as pl
from jax.experimental.pallas import tpu as pltpu

M = 65536
C = 1024
BM = 512
ALPHA = 0.7
BETA = 0.3
EPS = 1e-06
K = 13108


def _xla_rowsum_lanes(a):
    nr = a.shape[0]
    t = a[:, 0:128]
    for c in range(1, 8):
        t = t + a[:, 128 * c:128 * (c + 1)]
    zs = []
    for g in range(nr // 128):
        tt = t[128 * g:128 * (g + 1), :].T
        u = tt[0:8, :]
        for k in range(1, 16):
            u = u + tt[8 * k:8 * (k + 1), :]
        u = u[0:4, :] + u[4:8, :]
        u = u[0:2, :] + u[2:4, :]
        zs.append(u[0:1, :] + u[1:2, :])
    return jnp.concatenate(zs, axis=0)


def _rows_to_col(zl, nr):
    cols = [zl[g:g + 1, :].reshape(128, 1) for g in range(nr // 128)]
    return jnp.concatenate(cols, axis=0)


def _entropy_kernel(x_ref, out_ref):
    x = x_ref[...]
    x = jnp.where((x != x) | (jnp.abs(x) == jnp.inf), 0.0, x)
    m = jnp.max(x, axis=1, keepdims=True)
    e = jnp.exp(x - m)
    z = _rows_to_col(_xla_rowsum_lanes(e), BM)
    p = e / z
    lp = jnp.log(p + EPS)
    ent = -_xla_rowsum_lanes(p * lp)
    out_ref[0, 0, :] = ent.reshape(BM)


ENT_SCALE = float(np.float32(np.float32(ALPHA) / np.float32(math.log(C) + EPS)))


def _combine_kernel(ent_ref, geo_ref, out_ref):
    g = geo_ref[...]
    g = jnp.nan_to_num(g, nan=0.0, posinf=0.0, neginf=0.0)
    gmin = jnp.min(g)
    gmax = jnp.max(g)
    gn = (g - gmin) / (gmax - gmin + EPS)
    out_ref[...] = ent_ref[...] * ENT_SCALE + BETA * gn


def _scores2d(coarse_logits, handcrafted_features):
    nb = M // BM
    ent = pl.pallas_call(
        _entropy_kernel,
        grid=(nb,),
        in_specs=[pl.BlockSpec((BM, C), lambda i: (i, 0))],
        out_specs=pl.BlockSpec((1, 1, BM), lambda i: (i, 0, 0)),
        out_shape=jax.ShapeDtypeStruct((nb, 1, BM), jnp.float32),
    )(coarse_logits)
    ent2d = ent.reshape(M // 128, 128)
    geo2d = handcrafted_features[:, 2].reshape(M // 128, 128)
    total = pl.pallas_call(
        _combine_kernel,
        in_specs=[
            pl.BlockSpec((M // 128, 128), lambda: (0, 0)),
            pl.BlockSpec((M // 128, 128), lambda: (0, 0)),
        ],
        out_specs=pl.BlockSpec((M // 128, 128), lambda: (0, 0)),
        out_shape=jax.ShapeDtypeStruct((M // 128, 128), jnp.float32),
    )(ent2d, geo2d)
    return total


SR = 512
SC_ = 128
KROWS = 104


def _sort_kernel(sc_ref, idx_out_ref, key_scr, idx_scr):
    i = pl.program_id(0)
    r = jax.lax.broadcasted_iota(jnp.int32, (SR, SC_), 0)
    c = jax.lax.broadcasted_iota(jnp.int32, (SR, SC_), 1)

    @pl.when(i == 0)
    def _init():
        key_scr[...] = sc_ref[...]
        idx_scr[...] = r * SC_ + c

    k = jnp.int32(2) << i
    kr = k >> 7
    asc = ((c & k) == 0) & ((r & kr) == 0)

    for t in range(15, -1, -1):
        j = 1 << t

        @pl.when(2 * j <= k)
        def _stage(j=j):
            mk = key_scr[...]
            mi = idx_scr[...]
            if j >= SC_:
                d = j // SC_
                mybit = (r & d) != 0
                pk = jnp.where(mybit,
                               pltpu.roll(mk, d, axis=0),
                               pltpu.roll(mk, SR - d, axis=0))
                pi = jnp.where(mybit,
                               pltpu.roll(mi, d, axis=0),
                               pltpu.roll(mi, SR - d, axis=0))
            else:
                mybit = (c & j) != 0
                pk = jnp.where(mybit,
                               pltpu.roll(mk, j, axis=1),
                               pltpu.roll(mk, SC_ - j, axis=1))
                pi = jnp.where(mybit,
                               pltpu.roll(mi, j, axis=1),
                               pltpu.roll(mi, SC_ - j, axis=1))
            less_mp = (mk > pk) | ((mk == pk) & (mi < pi))
            sel_mine = ((asc != mybit) == less_mp)
            key_scr[...] = jnp.where(sel_mine, mk, pk)
            idx_scr[...] = jnp.where(sel_mine, mi, pi)

    @pl.when(i == 15)
    def _emit():
        idx_out_ref[...] = idx_scr[0:KROWS, :]


def _topk_sort(scores2d):
    idx = pl.pallas_call(
        _sort_kernel,
        grid=(16,),
        in_specs=[pl.BlockSpec((SR, SC_), lambda i: (0, 0))],
        out_specs=pl.BlockSpec((KROWS, SC_), lambda i: (0, 0)),
        out_shape=jax.ShapeDtypeStruct((KROWS, SC_), jnp.int32),
        scratch_shapes=[
            pltpu.VMEM((SR, SC_), jnp.float32),
            pltpu.VMEM((SR, SC_), jnp.int32),
        ],
    )(scores2d)
    return idx.reshape(KROWS * SC_)[:K]


def kernel(coarse_logits, handcrafted_features):
    total2d = _scores2d(coarse_logits, handcrafted_features)
    hard_sp_indices = _topk_sort(total2d)
    return (hard_sp_indices, total2d.reshape(M))


# speedup vs baseline: 2.5607x; 1.1401x over previous
import math

import numpy as np
import jax
import jax.numpy as jnp
from jax.experimental import pallas as pl
from jax.experimental.pallas import tpu as pltpu

M = 65536
C = 1024
BM = 1024
ALPHA = 0.7
BETA = 0.3
EPS = 1e-06
K = 13108


def _xla_rowsum_lanes(a):
    nr = a.shape[0]
    t = a[:, 0:128]
    for c in range(1, 8):
        t = t + a[:, 128 * c:128 * (c + 1)]
    zs = []
    for g in range(nr // 128):
        tt = t[128 * g:128 * (g + 1), :].T
        u = tt[0:8, :]
        for k in range(1, 16):
            u = u + tt[8 * k:8 * (k + 1), :]
        u = u[0:4, :] + u[4:8, :]
        u = u[0:2, :] + u[2:4, :]
        zs.append(u[0:1, :] + u[1:2, :])
    return jnp.concatenate(zs, axis=0)


def _rows_to_col(zl, nr):
    cols = [zl[g:g + 1, :].reshape(128, 1) for g in range(nr // 128)]
    return jnp.concatenate(cols, axis=0)


def _entropy_kernel(x_ref, out_ref):
    x = x_ref[...]
    x = jnp.where((x != x) | (jnp.abs(x) == jnp.inf), 0.0, x)
    m = jnp.max(x, axis=1, keepdims=True)
    e = jnp.exp(x - m)
    z = _rows_to_col(_xla_rowsum_lanes(e), BM)
    p = e / z
    lp = jnp.log(p + EPS)
    ent = -_xla_rowsum_lanes(p * lp)
    out_ref[0, 0, :] = ent.reshape(BM)


ENT_SCALE = float(np.float32(np.float32(ALPHA) / np.float32(math.log(C) + EPS)))


def _combine_kernel(ent_ref, geo_ref, out_ref):
    g = geo_ref[...]
    g = jnp.nan_to_num(g, nan=0.0, posinf=0.0, neginf=0.0)
    gmin = jnp.min(g)
    gmax = jnp.max(g)
    gn = (g - gmin) / (gmax - gmin + EPS)
    out_ref[...] = ent_ref[...] * ENT_SCALE + BETA * gn


def _scores2d(coarse_logits, handcrafted_features):
    nb = M // BM
    ent = pl.pallas_call(
        _entropy_kernel,
        grid=(nb,),
        in_specs=[pl.BlockSpec((BM, C), lambda i: (i, 0))],
        out_specs=pl.BlockSpec((1, 1, BM), lambda i: (i, 0, 0)),
        out_shape=jax.ShapeDtypeStruct((nb, 1, BM), jnp.float32),
    )(coarse_logits)
    ent2d = ent.reshape(M // 128, 128)
    geo2d = handcrafted_features[:, 2].reshape(M // 128, 128)
    total = pl.pallas_call(
        _combine_kernel,
        in_specs=[
            pl.BlockSpec((M // 128, 128), lambda: (0, 0)),
            pl.BlockSpec((M // 128, 128), lambda: (0, 0)),
        ],
        out_specs=pl.BlockSpec((M // 128, 128), lambda: (0, 0)),
        out_shape=jax.ShapeDtypeStruct((M // 128, 128), jnp.float32),
    )(ent2d, geo2d)
    return total


SR = 512
SC_ = 128
KROWS = 104


def _sort_kernel(sc_ref, idx_out_ref, key_scr, idx_scr):
    i = pl.program_id(0)
    r = jax.lax.broadcasted_iota(jnp.int32, (SR, SC_), 0)
    c = jax.lax.broadcasted_iota(jnp.int32, (SR, SC_), 1)

    @pl.when(i == 0)
    def _init():
        key_scr[...] = sc_ref[...]
        idx_scr[...] = r * SC_ + c

    k = jnp.int32(2) << i
    kr = k >> 7
    asc = ((c & k) == 0) & ((r & kr) == 0)

    for t in range(15, -1, -1):
        j = 1 << t

        @pl.when(2 * j <= k)
        def _stage(j=j):
            mk = key_scr[...]
            mi = idx_scr[...]
            if j >= SC_:
                d = j // SC_
                mybit = (r & d) != 0
                pk = jnp.where(mybit,
                               pltpu.roll(mk, d, axis=0),
                               pltpu.roll(mk, SR - d, axis=0))
                pi = jnp.where(mybit,
                               pltpu.roll(mi, d, axis=0),
                               pltpu.roll(mi, SR - d, axis=0))
            else:
                mybit = (c & j) != 0
                pk = jnp.where(mybit,
                               pltpu.roll(mk, j, axis=1),
                               pltpu.roll(mk, SC_ - j, axis=1))
                pi = jnp.where(mybit,
                               pltpu.roll(mi, j, axis=1),
                               pltpu.roll(mi, SC_ - j, axis=1))
            less_mp = (mk > pk) | ((mk == pk) & (mi < pi))
            sel_mine = ((asc != mybit) == less_mp)
            key_scr[...] = jnp.where(sel_mine, mk, pk)
            idx_scr[...] = jnp.where(sel_mine, mi, pi)

    @pl.when(i == 15)
    def _emit():
        idx_out_ref[...] = idx_scr[0:KROWS, :]


def _topk_sort(scores2d):
    idx = pl.pallas_call(
        _sort_kernel,
        grid=(16,),
        in_specs=[pl.BlockSpec((SR, SC_), lambda i: (0, 0))],
        out_specs=pl.BlockSpec((KROWS, SC_), lambda i: (0, 0)),
        out_shape=jax.ShapeDtypeStruct((KROWS, SC_), jnp.int32),
        scratch_shapes=[
            pltpu.VMEM((SR, SC_), jnp.float32),
            pltpu.VMEM((SR, SC_), jnp.int32),
        ],
    )(scores2d)
    return idx.reshape(KROWS * SC_)[:K]


def kernel(coarse_logits, handcrafted_features):
    total2d = _scores2d(coarse_logits, handcrafted_features)
    hard_sp_indices = _topk_sort(total2d)
    return (hard_sp_indices, total2d.reshape(M))


# speedup vs baseline: 2.6523x; 1.0358x over previous
import math

import numpy as np
import jax
import jax.numpy as jnp
from jax.experimental import pallas as pl
from jax.experimental.pallas import tpu as pltpu

M = 65536
C = 1024
BM = 2048
ALPHA = 0.7
BETA = 0.3
EPS = 1e-06
K = 13108


def _xla_rowsum_lanes(a):
    nr = a.shape[0]
    t = a[:, 0:128]
    for c in range(1, 8):
        t = t + a[:, 128 * c:128 * (c + 1)]
    zs = []
    for g in range(nr // 128):
        tt = t[128 * g:128 * (g + 1), :].T
        u = tt[0:8, :]
        for k in range(1, 16):
            u = u + tt[8 * k:8 * (k + 1), :]
        u = u[0:4, :] + u[4:8, :]
        u = u[0:2, :] + u[2:4, :]
        zs.append(u[0:1, :] + u[1:2, :])
    return jnp.concatenate(zs, axis=0)


def _rows_to_col(zl, nr):
    cols = [zl[g:g + 1, :].reshape(128, 1) for g in range(nr // 128)]
    return jnp.concatenate(cols, axis=0)


def _entropy_kernel(x_ref, out_ref):
    x = x_ref[...]
    x = jnp.where((x != x) | (jnp.abs(x) == jnp.inf), 0.0, x)
    m = jnp.max(x, axis=1, keepdims=True)
    e = jnp.exp(x - m)
    z = _rows_to_col(_xla_rowsum_lanes(e), BM)
    p = e / z
    lp = jnp.log(p + EPS)
    ent = -_xla_rowsum_lanes(p * lp)
    out_ref[0, 0, :] = ent.reshape(BM)


ENT_SCALE = float(np.float32(np.float32(ALPHA) / np.float32(math.log(C) + EPS)))


def _combine_kernel(ent_ref, geo_ref, out_ref):
    g = geo_ref[...]
    g = jnp.nan_to_num(g, nan=0.0, posinf=0.0, neginf=0.0)
    gmin = jnp.min(g)
    gmax = jnp.max(g)
    gn = (g - gmin) / (gmax - gmin + EPS)
    out_ref[...] = ent_ref[...] * ENT_SCALE + BETA * gn


def _scores2d(coarse_logits, handcrafted_features):
    nb = M // BM
    ent = pl.pallas_call(
        _entropy_kernel,
        grid=(nb,),
        in_specs=[pl.BlockSpec((BM, C), lambda i: (i, 0))],
        out_specs=pl.BlockSpec((1, 1, BM), lambda i: (i, 0, 0)),
        out_shape=jax.ShapeDtypeStruct((nb, 1, BM), jnp.float32),
    )(coarse_logits)
    ent2d = ent.reshape(M // 128, 128)
    geo2d = handcrafted_features[:, 2].reshape(M // 128, 128)
    total = pl.pallas_call(
        _combine_kernel,
        in_specs=[
            pl.BlockSpec((M // 128, 128), lambda: (0, 0)),
            pl.BlockSpec((M // 128, 128), lambda: (0, 0)),
        ],
        out_specs=pl.BlockSpec((M // 128, 128), lambda: (0, 0)),
        out_shape=jax.ShapeDtypeStruct((M // 128, 128), jnp.float32),
    )(ent2d, geo2d)
    return total


SR = 512
SC_ = 128
KROWS = 104


def _sort_kernel(sc_ref, idx_out_ref, key_scr, idx_scr):
    i = pl.program_id(0)
    r = jax.lax.broadcasted_iota(jnp.int32, (SR, SC_), 0)
    c = jax.lax.broadcasted_iota(jnp.int32, (SR, SC_), 1)

    @pl.when(i == 0)
    def _init():
        key_scr[...] = sc_ref[...]
        idx_scr[...] = r * SC_ + c

    k = jnp.int32(2) << i
    kr = k >> 7
    asc = ((c & k) == 0) & ((r & kr) == 0)

    for t in range(15, -1, -1):
        j = 1 << t

        @pl.when(2 * j <= k)
        def _stage(j=j):
            mk = key_scr[...]
            mi = idx_scr[...]
            if j >= SC_:
                d = j // SC_
                mybit = (r & d) != 0
                pk = jnp.where(mybit,
                               pltpu.roll(mk, d, axis=0),
                               pltpu.roll(mk, SR - d, axis=0))
                pi = jnp.where(mybit,
                               pltpu.roll(mi, d, axis=0),
                               pltpu.roll(mi, SR - d, axis=0))
            else:
                mybit = (c & j) != 0
                pk = jnp.where(mybit,
                               pltpu.roll(mk, j, axis=1),
                               pltpu.roll(mk, SC_ - j, axis=1))
                pi = jnp.where(mybit,
                               pltpu.roll(mi, j, axis=1),
                               pltpu.roll(mi, SC_ - j, axis=1))
            less_mp = (mk > pk) | ((mk == pk) & (mi < pi))
            sel_mine = ((asc != mybit) == less_mp)
            key_scr[...] = jnp.where(sel_mine, mk, pk)
            idx_scr[...] = jnp.where(sel_mine, mi, pi)

    @pl.when(i == 15)
    def _emit():
        idx_out_ref[...] = idx_scr[0:KROWS, :]


def _topk_sort(scores2d):
    idx = pl.pallas_call(
        _sort_kernel,
        grid=(16,),
        in_specs=[pl.BlockSpec((SR, SC_), lambda i: (0, 0))],
        out_specs=pl.BlockSpec((KROWS, SC_), lambda i: (0, 0)),
        out_shape=jax.ShapeDtypeStruct((KROWS, SC_), jnp.int32),
        scratch_shapes=[
            pltpu.VMEM((SR, SC_), jnp.float32),
            pltpu.VMEM((SR, SC_), jnp.int32),
        ],
    )(scores2d)
    return idx.reshape(KROWS * SC_)[:K]


def kernel(coarse_logits, handcrafted_features):
    total2d = _scores2d(coarse_logits, handcrafted_features)
    hard_sp_indices = _topk_sort(total2d)
    return (hard_sp_indices, total2d.reshape(M))


# speedup vs baseline: 2.6742x; 1.0083x over previous
import math

import numpy as np
import jax
import jax.numpy as jnp
from jax.experimental import pallas as pl
from jax.experimental.pallas import tpu as pltpu

M = 65536
C = 1024
BM = 2048
ALPHA = 0.7
BETA = 0.3
EPS = 1e-06
K = 13108


def _xla_rowsum_lanes(a):
    nr = a.shape[0]
    t = a[:, 0:128]
    for c in range(1, 8):
        t = t + a[:, 128 * c:128 * (c + 1)]
    zs = []
    for g in range(nr // 128):
        tt = t[128 * g:128 * (g + 1), :].T
        u = tt[0:8, :]
        for k in range(1, 16):
            u = u + tt[8 * k:8 * (k + 1), :]
        u = u[0:4, :] + u[4:8, :]
        u = u[0:2, :] + u[2:4, :]
        zs.append(u[0:1, :] + u[1:2, :])
    return jnp.concatenate(zs, axis=0)


def _rows_to_col(zl, nr):
    cols = [zl[g:g + 1, :].reshape(128, 1) for g in range(nr // 128)]
    return jnp.concatenate(cols, axis=0)


def _entropy_kernel(x_ref, out_ref):
    x = x_ref[...]
    x = jnp.where((x != x) | (jnp.abs(x) == jnp.inf), 0.0, x)
    m = jnp.max(x, axis=1, keepdims=True)
    e = jnp.exp(x - m)
    z = _rows_to_col(_xla_rowsum_lanes(e), BM)
    p = e / z
    lp = jnp.log(p + EPS)
    ent = -_xla_rowsum_lanes(p * lp)
    out_ref[0, 0, :] = ent.reshape(BM)


ENT_SCALE = float(np.float32(np.float32(ALPHA) / np.float32(math.log(C) + EPS)))


def _entropy(coarse_logits):
    nb = M // BM
    ent = pl.pallas_call(
        _entropy_kernel,
        grid=(nb,),
        in_specs=[pl.BlockSpec((BM, C), lambda i: (i, 0))],
        out_specs=pl.BlockSpec((1, 1, BM), lambda i: (i, 0, 0)),
        out_shape=jax.ShapeDtypeStruct((nb, 1, BM), jnp.float32),
    )(coarse_logits)
    return ent.reshape(M // 128, 128)


SR = 512
SC_ = 128
KROWS = 104


def _sort_kernel(ent_ref, geo_ref, sc_out_ref, idx_out_ref, key_scr, idx_scr):
    i = pl.program_id(0)
    r = jax.lax.broadcasted_iota(jnp.int32, (SR, SC_), 0)
    c = jax.lax.broadcasted_iota(jnp.int32, (SR, SC_), 1)

    @pl.when(i == 0)
    def _init():
        g = geo_ref[...]
        g = jnp.nan_to_num(g, nan=0.0, posinf=0.0, neginf=0.0)
        gmin = jnp.min(g)
        gmax = jnp.max(g)
        gn = (g - gmin) / (gmax - gmin + EPS)
        total = ent_ref[...] * ENT_SCALE + BETA * gn
        sc_out_ref[...] = total
        key_scr[...] = total
        idx_scr[...] = r * SC_ + c

    k = jnp.int32(2) << i
    kr = k >> 7
    asc = ((c & k) == 0) & ((r & kr) == 0)

    for t in range(15, -1, -1):
        j = 1 << t

        @pl.when(2 * j <= k)
        def _stage(j=j):
            mk = key_scr[...]
            mi = idx_scr[...]
            if j >= SC_:
                d = j // SC_
                mybit = (r & d) != 0
                pk = jnp.where(mybit,
                               pltpu.roll(mk, d, axis=0),
                               pltpu.roll(mk, SR - d, axis=0))
                pi = jnp.where(mybit,
                               pltpu.roll(mi, d, axis=0),
                               pltpu.roll(mi, SR - d, axis=0))
            else:
                mybit = (c & j) != 0
                pk = jnp.where(mybit,
                               pltpu.roll(mk, j, axis=1),
                               pltpu.roll(mk, SC_ - j, axis=1))
                pi = jnp.where(mybit,
                               pltpu.roll(mi, j, axis=1),
                               pltpu.roll(mi, SC_ - j, axis=1))
            less_mp = (mk > pk) | ((mk == pk) & (mi < pi))
            sel_mine = ((asc != mybit) == less_mp)
            key_scr[...] = jnp.where(sel_mine, mk, pk)
            idx_scr[...] = jnp.where(sel_mine, mi, pi)

    @pl.when(i == 15)
    def _emit():
        idx_out_ref[...] = idx_scr[0:KROWS, :]


def _combine_topk_sort(ent2d, geo2d):
    total2d, idx = pl.pallas_call(
        _sort_kernel,
        grid=(16,),
        in_specs=[
            pl.BlockSpec((SR, SC_), lambda i: (0, 0)),
            pl.BlockSpec((SR, SC_), lambda i: (0, 0)),
        ],
        out_specs=[
            pl.BlockSpec((SR, SC_), lambda i: (0, 0)),
            pl.BlockSpec((KROWS, SC_), lambda i: (0, 0)),
        ],
        out_shape=[
            jax.ShapeDtypeStruct((SR, SC_), jnp.float32),
            jax.ShapeDtypeStruct((KROWS, SC_), jnp.int32),
        ],
        scratch_shapes=[
            pltpu.VMEM((SR, SC_), jnp.float32),
            pltpu.VMEM((SR, SC_), jnp.int32),
        ],
    )(ent2d, geo2d)
    return total2d, idx.reshape(KROWS * SC_)[:K]


def kernel(coarse_logits, handcrafted_features):
    ent2d = _entropy(coarse_logits)
    geo2d = handcrafted_features[:, 2].reshape(M // 128, 128)
    total2d, hard_sp_indices = _combine_topk_sort(ent2d, geo2d)
    return (hard_sp_indices, total2d.reshape(M))
